# Initial kernel scaffold; baseline (speedup 1.0000x reference)
import jax
import jax.numpy as jnp
from jax.experimental import pallas as pl


def kernel(embedding_weight, edge_index, edge_weight):
    raise NotImplementedError("write your pallas kernel here")


# speedup vs baseline: 6.1824x; 6.1824x over previous
import functools

import jax
import jax.numpy as jnp
from jax import lax
from jax.experimental import pallas as pl
from jax.experimental.pallas import tpu as pltpu
from jax.experimental.pallas import tpu_sc as plsc

N_USERS = 30000
N_ITEMS = 19000
N_GENRES = 1000
N_NODES = 50000
N_EDGES = 800000
D = 64
N_LAYERS = 3

NC = 2
NS = 16
L = 16
HALF = N_NODES // NC
TRASH = 600
R_ACC = HALF + TRASH

_MESH = dict(
    mesh=plsc.VectorSubcoreMesh(core_axis_name="c", subcore_axis_name="s"),
    compiler_params=pltpu.CompilerParams(
        needs_layout_passes=False, use_tc_tiling_on_sc=False),
)


def _rsqrt_newton(x):
    bits = lax.bitcast_convert_type(x, jnp.int32)
    y = lax.bitcast_convert_type(
        jnp.int32(0x5F3759DF) - lax.shift_right_logical(bits, 1), jnp.float32)
    for _ in range(3):
        y = y * (1.5 - 0.5 * x * y * y)
    return jnp.where(x > 0.0, y, 0.0)


_DEG_B = 80
_DEG_ROWS = 3136


@functools.partial(
    pl.kernel,
    out_type=(jax.ShapeDtypeStruct((N_NODES,), jnp.float32),
              jax.ShapeDtypeStruct((N_NODES,), jnp.float32)),
    scratch_types=[
        pltpu.VMEM_SHARED((N_NODES,), jnp.float32),
        pltpu.VMEM((_DEG_B,), jnp.int32),
        pltpu.VMEM((_DEG_B,), jnp.float32),
        pltpu.VMEM((_DEG_ROWS,), jnp.float32),
        pltpu.VMEM((_DEG_ROWS,), jnp.float32),
    ],
    **_MESH,
)
def _deg_kernel(src_hbm, dst_hbm, w_hbm, d0_hbm, d1_hbm, acc, idx_v, w_v,
                rd_v, out_v):
    c = lax.axis_index("c")
    t = lax.axis_index("s")
    zeros16 = jnp.zeros((L,), jnp.float32)

    def _z(i, _):
        rd_v[pl.ds(i * L, L)] = zeros16
        return _
    lax.fori_loop(0, _DEG_ROWS // L, _z, None)
    rbase = t * _DEG_ROWS

    @pl.when(t < NS - 1)
    def _z_full():
        pltpu.sync_copy(rd_v, acc.at[pl.ds(rbase, _DEG_ROWS)])

    @pl.when(t == NS - 1)
    def _z_tail():
        pltpu.sync_copy(rd_v.at[pl.ds(0, 2960)], acc.at[pl.ds(rbase, 2960)])
    plsc.subcore_barrier()

    ebase = t * (N_EDGES // NS)

    def _chunk(j, _):
        off = ebase + j * _DEG_B

        @pl.when(c == 0)
        def _ld0():
            pltpu.sync_copy(src_hbm.at[pl.ds(off, _DEG_B)], idx_v)

        @pl.when(c == 1)
        def _ld1():
            pltpu.sync_copy(dst_hbm.at[pl.ds(off, _DEG_B)], idx_v)
        pltpu.sync_copy(w_hbm.at[pl.ds(off, _DEG_B)], w_v)
        pltpu.sync_copy(w_v, acc.at[idx_v], add=True)
        return _
    lax.fori_loop(0, (N_EDGES // NS) // _DEG_B, _chunk, None)
    plsc.subcore_barrier()

    n_grp = jnp.where(t < NS - 1, _DEG_ROWS // L, 2960 // L)

    @pl.when(t < NS - 1)
    def _rd_full():
        pltpu.sync_copy(acc.at[pl.ds(rbase, _DEG_ROWS)], rd_v)

    @pl.when(t == NS - 1)
    def _rd_tail():
        pltpu.sync_copy(acc.at[pl.ds(rbase, 2960)], rd_v.at[pl.ds(0, 2960)])

    def _g(g, _):
        s = pl.ds(g * L, L)
        out_v[s] = _rsqrt_newton(rd_v[s])
        return _
    lax.fori_loop(0, n_grp, _g, None)

    @pl.when((t < NS - 1) & (c == 0))
    def _wr_full0():
        pltpu.sync_copy(out_v, d0_hbm.at[pl.ds(rbase, _DEG_ROWS)])

    @pl.when((t == NS - 1) & (c == 0))
    def _wr_tail0():
        pltpu.sync_copy(out_v.at[pl.ds(0, 2960)], d0_hbm.at[pl.ds(rbase, 2960)])

    @pl.when((t < NS - 1) & (c == 1))
    def _wr_full1():
        pltpu.sync_copy(out_v, d1_hbm.at[pl.ds(rbase, _DEG_ROWS)])

    @pl.when((t == NS - 1) & (c == 1))
    def _wr_tail1():
        pltpu.sync_copy(out_v.at[pl.ds(0, 2960)], d1_hbm.at[pl.ds(rbase, 2960)])


_NW_B = 128
_NW_PER = 25088


@functools.partial(
    pl.kernel,
    out_type=jax.ShapeDtypeStruct((N_EDGES,), jnp.float32),
    scratch_types=[
        pltpu.VMEM((N_NODES,), jnp.float32),
        pltpu.VMEM((N_NODES,), jnp.float32),
        pltpu.VMEM((_NW_B,), jnp.int32),
        pltpu.VMEM((_NW_B,), jnp.int32),
        pltpu.VMEM((_NW_B,), jnp.float32),
        pltpu.VMEM((_NW_B,), jnp.float32),
    ],
    **_MESH,
)
def _normw_kernel(src_hbm, dst_hbm, w_hbm, d0_hbm, d1_hbm, nw_hbm, d0_v,
                  d1_v, src_v, dst_v, w_v, o_v):
    wid = lax.axis_index("s") * NC + lax.axis_index("c")
    pltpu.sync_copy(d0_hbm, d0_v)
    pltpu.sync_copy(d1_hbm, d1_v)
    base = wid * _NW_PER
    n_chunks = jnp.where(wid < NC * NS - 1, _NW_PER // _NW_B, 22272 // _NW_B)

    def _chunk(j, _):
        off = base + j * _NW_B
        pltpu.sync_copy(src_hbm.at[pl.ds(off, _NW_B)], src_v)
        pltpu.sync_copy(dst_hbm.at[pl.ds(off, _NW_B)], dst_v)
        pltpu.sync_copy(w_hbm.at[pl.ds(off, _NW_B)], w_v)
        for k in range(_NW_B // L):
            s = pl.ds(k * L, L)
            a = plsc.load_gather(d0_v, [src_v[s]])
            b = plsc.load_gather(d1_v, [dst_v[s]])
            o_v[s] = a * w_v[s] * b
        pltpu.sync_copy(o_v, nw_hbm.at[pl.ds(off, _NW_B)])
        return _
    lax.fori_loop(0, n_chunks, _chunk, None)


_LY_B = 80
_LY_EB = 2000
_LY_PER_TILE = N_EDGES // NS


@functools.partial(
    pl.kernel,
    out_type=jax.ShapeDtypeStruct((N_NODES, D), jnp.float32),
    scratch_types=[
        pltpu.VMEM_SHARED((R_ACC, D), jnp.float32),
        pltpu.VMEM((_LY_EB,), jnp.int32),
        pltpu.VMEM((_LY_EB,), jnp.int32),
        pltpu.VMEM((_LY_EB,), jnp.float32),
        pltpu.VMEM((_LY_B, D), jnp.float32),
        pltpu.VMEM((_LY_B,), jnp.int32),
        pltpu.VMEM((40, D), jnp.float32),
        pltpu.VMEM((125, D), jnp.float32),
    ],
    **_MESH,
)
def _layer_kernel(emb_hbm, srce_hbm, dste_hbm, nw_hbm, out_hbm, acc, src_v,
                  dst_v, nw_v, rows_v, sidx_v, z_v, o_v):
    c = lax.axis_index("c")
    t = lax.axis_index("s")
    iota = lax.iota(jnp.int32, L)
    zeros16 = jnp.zeros((L,), jnp.float32)

    def _z(i, _):
        z_v[i % 40, pl.ds((i // 40) * L, L)] = zeros16
        return _
    lax.fori_loop(0, 40 * (D // L), _z, None)

    def _zacc(i, _):
        pltpu.sync_copy(z_v, acc.at[pl.ds(t * 1600 + i * 40, 40)])
        return _
    lax.fori_loop(0, 40, _zacc, None)
    plsc.subcore_barrier()

    lo = c * HALF

    def _outer(j, _):
        eb = t * _LY_PER_TILE + j * _LY_EB
        pltpu.sync_copy(srce_hbm.at[pl.ds(eb, _LY_EB)], src_v)
        pltpu.sync_copy(dste_hbm.at[pl.ds(eb, _LY_EB)], dst_v)
        pltpu.sync_copy(nw_hbm.at[pl.ds(eb, _LY_EB)], nw_v)

        def _inner(i, _):
            off = i * _LY_B
            pltpu.sync_copy(emb_hbm.at[dst_v.at[pl.ds(off, _LY_B)]], rows_v)

            def _scale(g, _):
                wv = nw_v[pl.ds(off + g * L, L)]
                for u in range(L):
                    w = wv[u]
                    e = g * L + u
                    for k in range(D // L):
                        s = pl.ds(k * L, L)
                        rows_v[e, s] = rows_v[e, s] * w
                return _
            lax.fori_loop(0, _LY_B // L, _scale, None)
            for k in range(_LY_B // L):
                s = pl.ds(k * L, L)
                sv = src_v[pl.ds(off + k * L, L)]
                loc = sv - lo
                m = (loc >= 0) & (loc < HALF)
                sidx_v[s] = jnp.where(m, loc, HALF + (sv & 511))
            pltpu.sync_copy(rows_v, acc.at[sidx_v], add=True)
            return _
        lax.fori_loop(0, _LY_EB // _LY_B, _inner, None)
        return _
    lax.fori_loop(0, _LY_PER_TILE // _LY_EB, _outer, None)
    plsc.subcore_barrier()

    n_i = jnp.where(t < 8, 13, 12)

    def _wb(i, _):
        q = t + i * NS
        pltpu.sync_copy(acc.at[pl.ds(q * 125, 125)], o_v)
        pltpu.sync_copy(o_v, out_hbm.at[pl.ds(c * HALF + q * 125, 125)])
        return _
    lax.fori_loop(0, n_i, _wb, None)


def _mean_body(a, b, c, d, o):
    o[...] = (a[...] + b[...] + c[...] + d[...]) * 0.25


def _mean4(e0, e1, e2, e3):
    blk = pl.BlockSpec((2000, D), lambda i: (i, 0))
    return pl.pallas_call(
        _mean_body,
        grid=(N_NODES // 2000,),
        in_specs=[blk] * 4,
        out_specs=blk,
        out_shape=jax.ShapeDtypeStruct((N_NODES, D), jnp.float32),
    )(e0, e1, e2, e3)


def kernel(embedding_weight, edge_index, edge_weight):
    src = edge_index[0]
    dst = edge_index[1]
    d0, d1 = _deg_kernel(src, dst, edge_weight)
    nw = _normw_kernel(src, dst, edge_weight, d0, d1)
    emb = embedding_weight
    embs = [emb]
    for _ in range(N_LAYERS):
        emb = _layer_kernel(emb, src, dst, nw)
        embs.append(emb)
    return _mean4(*embs)


# speedup vs baseline: 8.7513x; 1.4155x over previous
import functools

import jax
import jax.numpy as jnp
from jax import lax
from jax.experimental import pallas as pl
from jax.experimental.pallas import tpu as pltpu
from jax.experimental.pallas import tpu_sc as plsc

N_USERS = 30000
N_ITEMS = 19000
N_GENRES = 1000
N_NODES = 50000
N_EDGES = 800000
D = 64
N_LAYERS = 3

NC = 2
NS = 16
L = 16
HALF = N_NODES // NC
TRASH = 600
R_ACC = HALF + TRASH

_MESH = dict(
    mesh=plsc.VectorSubcoreMesh(core_axis_name="c", subcore_axis_name="s"),
    compiler_params=pltpu.CompilerParams(
        needs_layout_passes=False, use_tc_tiling_on_sc=False),
)


def _rsqrt_newton(x):
    bits = lax.bitcast_convert_type(x, jnp.int32)
    y = lax.bitcast_convert_type(
        jnp.int32(0x5F3759DF) - lax.shift_right_logical(bits, 1), jnp.float32)
    for _ in range(3):
        y = y * (1.5 - 0.5 * x * y * y)
    return jnp.where(x > 0.0, y, 0.0)


_DEG_B = 80
_DEG_ROWS = 3136


@functools.partial(
    pl.kernel,
    out_type=(jax.ShapeDtypeStruct((N_NODES,), jnp.float32),
              jax.ShapeDtypeStruct((N_NODES,), jnp.float32)),
    scratch_types=[
        pltpu.VMEM_SHARED((N_NODES,), jnp.float32),
        pltpu.VMEM((_DEG_B,), jnp.int32),
        pltpu.VMEM((_DEG_B,), jnp.float32),
        pltpu.VMEM((_DEG_ROWS,), jnp.float32),
        pltpu.VMEM((_DEG_ROWS,), jnp.float32),
    ],
    **_MESH,
)
def _deg_kernel(src_hbm, dst_hbm, w_hbm, d0_hbm, d1_hbm, acc, idx_v, w_v,
                rd_v, out_v):
    c = lax.axis_index("c")
    t = lax.axis_index("s")
    zeros16 = jnp.zeros((L,), jnp.float32)

    def _z(i, _):
        rd_v[pl.ds(i * L, L)] = zeros16
        return _
    lax.fori_loop(0, _DEG_ROWS // L, _z, None)
    rbase = t * _DEG_ROWS

    @pl.when(t < NS - 1)
    def _z_full():
        pltpu.sync_copy(rd_v, acc.at[pl.ds(rbase, _DEG_ROWS)])

    @pl.when(t == NS - 1)
    def _z_tail():
        pltpu.sync_copy(rd_v.at[pl.ds(0, 2960)], acc.at[pl.ds(rbase, 2960)])
    plsc.subcore_barrier()

    ebase = t * (N_EDGES // NS)

    def _chunk(j, _):
        off = ebase + j * _DEG_B

        @pl.when(c == 0)
        def _ld0():
            pltpu.sync_copy(src_hbm.at[pl.ds(off, _DEG_B)], idx_v)

        @pl.when(c == 1)
        def _ld1():
            pltpu.sync_copy(dst_hbm.at[pl.ds(off, _DEG_B)], idx_v)
        pltpu.sync_copy(w_hbm.at[pl.ds(off, _DEG_B)], w_v)
        pltpu.sync_copy(w_v, acc.at[idx_v], add=True)
        return _
    lax.fori_loop(0, (N_EDGES // NS) // _DEG_B, _chunk, None)
    plsc.subcore_barrier()

    n_grp = jnp.where(t < NS - 1, _DEG_ROWS // L, 2960 // L)

    @pl.when(t < NS - 1)
    def _rd_full():
        pltpu.sync_copy(acc.at[pl.ds(rbase, _DEG_ROWS)], rd_v)

    @pl.when(t == NS - 1)
    def _rd_tail():
        pltpu.sync_copy(acc.at[pl.ds(rbase, 2960)], rd_v.at[pl.ds(0, 2960)])

    def _g(g, _):
        s = pl.ds(g * L, L)
        out_v[s] = _rsqrt_newton(rd_v[s])
        return _
    lax.fori_loop(0, n_grp, _g, None)

    @pl.when((t < NS - 1) & (c == 0))
    def _wr_full0():
        pltpu.sync_copy(out_v, d0_hbm.at[pl.ds(rbase, _DEG_ROWS)])

    @pl.when((t == NS - 1) & (c == 0))
    def _wr_tail0():
        pltpu.sync_copy(out_v.at[pl.ds(0, 2960)], d0_hbm.at[pl.ds(rbase, 2960)])

    @pl.when((t < NS - 1) & (c == 1))
    def _wr_full1():
        pltpu.sync_copy(out_v, d1_hbm.at[pl.ds(rbase, _DEG_ROWS)])

    @pl.when((t == NS - 1) & (c == 1))
    def _wr_tail1():
        pltpu.sync_copy(out_v.at[pl.ds(0, 2960)], d1_hbm.at[pl.ds(rbase, 2960)])


_NW_B = 128
_NW_PER = 25088


@functools.partial(
    pl.kernel,
    out_type=jax.ShapeDtypeStruct((N_EDGES,), jnp.float32),
    scratch_types=[
        pltpu.VMEM((N_NODES,), jnp.float32),
        pltpu.VMEM((N_NODES,), jnp.float32),
        pltpu.VMEM((_NW_B,), jnp.int32),
        pltpu.VMEM((_NW_B,), jnp.int32),
        pltpu.VMEM((_NW_B,), jnp.float32),
        pltpu.VMEM((_NW_B,), jnp.float32),
    ],
    **_MESH,
)
def _normw_kernel(src_hbm, dst_hbm, w_hbm, d0_hbm, d1_hbm, nw_hbm, d0_v,
                  d1_v, src_v, dst_v, w_v, o_v):
    wid = lax.axis_index("s") * NC + lax.axis_index("c")
    pltpu.sync_copy(d0_hbm, d0_v)
    pltpu.sync_copy(d1_hbm, d1_v)
    base = wid * _NW_PER
    n_chunks = jnp.where(wid < NC * NS - 1, _NW_PER // _NW_B, 22272 // _NW_B)

    def _chunk(j, _):
        off = base + j * _NW_B
        pltpu.sync_copy(src_hbm.at[pl.ds(off, _NW_B)], src_v)
        pltpu.sync_copy(dst_hbm.at[pl.ds(off, _NW_B)], dst_v)
        pltpu.sync_copy(w_hbm.at[pl.ds(off, _NW_B)], w_v)
        for k in range(_NW_B // L):
            s = pl.ds(k * L, L)
            a = plsc.load_gather(d0_v, [src_v[s]])
            b = plsc.load_gather(d1_v, [dst_v[s]])
            o_v[s] = a * w_v[s] * b
        pltpu.sync_copy(o_v, nw_hbm.at[pl.ds(off, _NW_B)])
        return _
    lax.fori_loop(0, n_chunks, _chunk, None)


_LY_B = 80
_LY_EB = 2000
_LY_PER_TILE = N_EDGES // NS
_NQ_FULL = HALF // _LY_B


@functools.partial(
    pl.kernel,
    out_type=jax.ShapeDtypeStruct((N_NODES, D), jnp.float32),
    scratch_types=[
        pltpu.VMEM_SHARED((HALF, D), jnp.float32),
        pltpu.VMEM((_LY_EB,), jnp.int32),
        pltpu.VMEM((_LY_EB,), jnp.int32),
        pltpu.VMEM((_LY_EB,), jnp.float32),
        pltpu.VMEM((_LY_B, D), jnp.float32),
        pltpu.VMEM((_LY_B, D), jnp.float32),
        pltpu.VMEM((_LY_B, D), jnp.float32),
        pltpu.VMEM((_LY_B, D), jnp.float32),
        pltpu.VMEM((_LY_B,), jnp.int32),
        pltpu.VMEM((_LY_B,), jnp.int32),
        pltpu.SemaphoreType.DMA,
        pltpu.SemaphoreType.DMA,
        pltpu.SemaphoreType.DMA,
        pltpu.SemaphoreType.DMA,
    ],
    **_MESH,
)
def _layer_kernel(emb_hbm, srce_hbm, dste_hbm, nw_hbm, out_hbm, acc, src_v,
                  dst_v, nw_v, gb0, gb1, sb0, sb1, sx0, sx1, gsem0, gsem1,
                  ssem0, ssem1):
    c = lax.axis_index("c")
    t = lax.axis_index("s")
    lo = c * HALF
    zeros16 = jnp.zeros((L,), jnp.float32)

    for k in range(D // L):
        def _z(i, _, k=k):
            sb0[i, pl.ds(k * L, L)] = zeros16
            return _
        lax.fori_loop(0, _LY_B, _z, None)
    n_q = jnp.where(t < 9, 20, 19)

    def _zacc(i, _):
        q = t + i * NS

        @pl.when(q < _NQ_FULL)
        def _full():
            pltpu.sync_copy(sb0, acc.at[pl.ds(q * _LY_B, _LY_B)])

        @pl.when(q == _NQ_FULL)
        def _tail():
            pltpu.sync_copy(sb0.at[pl.ds(0, 40)],
                            acc.at[pl.ds(q * _LY_B, 40)])
        return _
    lax.fori_loop(0, n_q, _zacc, None)
    plsc.subcore_barrier()

    def _issue_g(off, gb, gsem):
        pltpu.async_copy(emb_hbm.at[dst_v.at[pl.ds(off, _LY_B)]], gb, gsem)

    def _wait_g(off, gb, gsem):
        pltpu.make_async_copy(emb_hbm.at[dst_v.at[pl.ds(off, _LY_B)]], gb,
                              gsem).wait()

    def _issue_s(sb, sx, ssem):
        pltpu.async_copy(sb, acc.at[sx], ssem, add=True)

    def _wait_s(sb, sx, ssem):
        pltpu.make_async_copy(sb, acc.at[sx], ssem).wait()

    def _scale_chunk(off, gb, sb, sx):
        for g in range(_LY_B // L):
            s16 = pl.ds(off + g * L, L)
            sv = src_v[s16]
            loc = sv - lo
            m = (loc >= 0) & (loc < HALF)
            sx[pl.ds(g * L, L)] = jnp.where(m, loc, sv & 16383)
            wv = jnp.where(m, nw_v[s16], 0.0)
            for u in range(L):
                w = wv[u]
                e = g * L + u
                for k in range(D // L):
                    ss = pl.ds(k * L, L)
                    sb[e, ss] = gb[e, ss] * w

    def _block(bj, _):
        eb = t * _LY_PER_TILE + bj * _LY_EB
        pltpu.sync_copy(srce_hbm.at[pl.ds(eb, _LY_EB)], src_v)
        pltpu.sync_copy(dste_hbm.at[pl.ds(eb, _LY_EB)], dst_v)
        pltpu.sync_copy(nw_hbm.at[pl.ds(eb, _LY_EB)], nw_v)
        _issue_g(0, gb0, gsem0)

        def _pair(i, _):
            o0 = 2 * i * _LY_B
            o1 = o0 + _LY_B
            o2 = o1 + _LY_B
            _wait_g(o0, gb0, gsem0)
            _issue_g(o1, gb1, gsem1)

            @pl.when(i >= 1)
            def _w0():
                _wait_s(sb0, sx0, ssem0)
            _scale_chunk(o0, gb0, sb0, sx0)
            _issue_s(sb0, sx0, ssem0)

            _wait_g(o1, gb1, gsem1)
            _issue_g(o2, gb0, gsem0)

            @pl.when(i >= 1)
            def _w1():
                _wait_s(sb1, sx1, ssem1)
            _scale_chunk(o1, gb1, sb1, sx1)
            _issue_s(sb1, sx1, ssem1)
            return _
        lax.fori_loop(0, 12, _pair, None)

        o24 = 24 * _LY_B
        _wait_g(o24, gb0, gsem0)
        _wait_s(sb0, sx0, ssem0)
        _scale_chunk(o24, gb0, sb0, sx0)
        _issue_s(sb0, sx0, ssem0)
        _wait_s(sb1, sx1, ssem1)
        _wait_s(sb0, sx0, ssem0)
        return _
    lax.fori_loop(0, _LY_PER_TILE // _LY_EB, _block, None)
    plsc.subcore_barrier()

    def _wb(i, _):
        q = t + i * NS

        @pl.when(q < _NQ_FULL)
        def _full():
            pltpu.sync_copy(acc.at[pl.ds(q * _LY_B, _LY_B)], gb0)
            pltpu.sync_copy(gb0, out_hbm.at[pl.ds(c * HALF + q * _LY_B,
                                                  _LY_B)])

        @pl.when(q == _NQ_FULL)
        def _tail():
            pltpu.sync_copy(acc.at[pl.ds(q * _LY_B, 40)],
                            gb0.at[pl.ds(0, 40)])
            pltpu.sync_copy(gb0.at[pl.ds(0, 40)],
                            out_hbm.at[pl.ds(c * HALF + q * _LY_B, 40)])
        return _
    lax.fori_loop(0, n_q, _wb, None)


def _mean_body(a, b, c, d, o):
    o[...] = (a[...] + b[...] + c[...] + d[...]) * 0.25


def _mean4(e0, e1, e2, e3):
    blk = pl.BlockSpec((2000, D), lambda i: (i, 0))
    return pl.pallas_call(
        _mean_body,
        grid=(N_NODES // 2000,),
        in_specs=[blk] * 4,
        out_specs=blk,
        out_shape=jax.ShapeDtypeStruct((N_NODES, D), jnp.float32),
    )(e0, e1, e2, e3)


def kernel(embedding_weight, edge_index, edge_weight):
    src = edge_index[0]
    dst = edge_index[1]
    d0, d1 = _deg_kernel(src, dst, edge_weight)
    nw = _normw_kernel(src, dst, edge_weight, d0, d1)
    emb = embedding_weight
    embs = [emb]
    for _ in range(N_LAYERS):
        emb = _layer_kernel(emb, src, dst, nw)
        embs.append(emb)
    return _mean4(*embs)


# speedup vs baseline: 12.1951x; 1.3935x over previous
import functools

import jax
import jax.numpy as jnp
from jax import lax
from jax.experimental import pallas as pl
from jax.experimental.pallas import tpu as pltpu
from jax.experimental.pallas import tpu_sc as plsc

N_USERS = 30000
N_ITEMS = 19000
N_GENRES = 1000
N_NODES = 50000
N_EDGES = 800000
D = 64
N_LAYERS = 3

NC = 2
NS = 16
L = 16
HALF = N_NODES // NC
TRASH = 600
R_ACC = HALF + TRASH

_MESH = dict(
    mesh=plsc.VectorSubcoreMesh(core_axis_name="c", subcore_axis_name="s"),
    compiler_params=pltpu.CompilerParams(
        needs_layout_passes=False, use_tc_tiling_on_sc=False),
)


def _rsqrt_newton(x):
    bits = lax.bitcast_convert_type(x, jnp.int32)
    y = lax.bitcast_convert_type(
        jnp.int32(0x5F3759DF) - lax.shift_right_logical(bits, 1), jnp.float32)
    for _ in range(3):
        y = y * (1.5 - 0.5 * x * y * y)
    return jnp.where(x > 0.0, y, 0.0)


_DEG_B = 80
_DEG_RPB = 25
_DEG_ROWS = 3136


@functools.partial(
    pl.kernel,
    out_type=(jax.ShapeDtypeStruct((N_NODES,), jnp.float32),
              jax.ShapeDtypeStruct((N_NODES,), jnp.float32)),
    scratch_types=[
        pltpu.VMEM_SHARED((N_NODES,), jnp.float32),
        pltpu.VMEM((_DEG_RPB, _DEG_B), jnp.int32),
        pltpu.VMEM((_DEG_RPB, _DEG_B), jnp.float32),
        pltpu.VMEM((_DEG_ROWS,), jnp.float32),
        pltpu.VMEM((_DEG_ROWS,), jnp.float32),
        pltpu.SemaphoreType.DMA,
    ],
    **_MESH,
)
def _deg_kernel(srcr_hbm, dstr_hbm, wr_hbm, d0_hbm, d1_hbm, acc, idx_v, w_v,
                rd_v, out_v, ssem):
    c = lax.axis_index("c")
    t = lax.axis_index("s")
    zeros16 = jnp.zeros((L,), jnp.float32)

    def _z(i, _):
        rd_v[pl.ds(i * L, L)] = zeros16
        return _
    lax.fori_loop(0, _DEG_ROWS // L, _z, None)
    rbase = t * _DEG_ROWS

    @pl.when(t < NS - 1)
    def _z_full():
        pltpu.sync_copy(rd_v, acc.at[pl.ds(rbase, _DEG_ROWS)])

    @pl.when(t == NS - 1)
    def _z_tail():
        pltpu.sync_copy(rd_v.at[pl.ds(0, 2960)], acc.at[pl.ds(rbase, 2960)])
    plsc.subcore_barrier()

    rows_per_tile = N_EDGES // NS // _DEG_B

    def _block(b, _):
        row0 = t * rows_per_tile + b * _DEG_RPB

        @pl.when(c == 0)
        def _ld0():
            pltpu.sync_copy(srcr_hbm.at[pl.ds(row0, _DEG_RPB)], idx_v)

        @pl.when(c == 1)
        def _ld1():
            pltpu.sync_copy(dstr_hbm.at[pl.ds(row0, _DEG_RPB)], idx_v)
        pltpu.sync_copy(wr_hbm.at[pl.ds(row0, _DEG_RPB)], w_v)
        for j in range(_DEG_RPB):
            pltpu.async_copy(w_v.at[j], acc.at[idx_v.at[j]], ssem, add=True)
        for j in range(_DEG_RPB):
            pltpu.make_async_copy(w_v.at[j], acc.at[idx_v.at[j]], ssem).wait()
        return _
    lax.fori_loop(0, rows_per_tile // _DEG_RPB, _block, None)
    plsc.subcore_barrier()

    n_grp = jnp.where(t < NS - 1, _DEG_ROWS // L, 2960 // L)

    @pl.when(t < NS - 1)
    def _rd_full():
        pltpu.sync_copy(acc.at[pl.ds(rbase, _DEG_ROWS)], rd_v)

    @pl.when(t == NS - 1)
    def _rd_tail():
        pltpu.sync_copy(acc.at[pl.ds(rbase, 2960)], rd_v.at[pl.ds(0, 2960)])

    def _g(g, _):
        s = pl.ds(g * L, L)
        out_v[s] = _rsqrt_newton(rd_v[s])
        return _
    lax.fori_loop(0, n_grp, _g, None)

    @pl.when((t < NS - 1) & (c == 0))
    def _wr_full0():
        pltpu.sync_copy(out_v, d0_hbm.at[pl.ds(rbase, _DEG_ROWS)])

    @pl.when((t == NS - 1) & (c == 0))
    def _wr_tail0():
        pltpu.sync_copy(out_v.at[pl.ds(0, 2960)], d0_hbm.at[pl.ds(rbase, 2960)])

    @pl.when((t < NS - 1) & (c == 1))
    def _wr_full1():
        pltpu.sync_copy(out_v, d1_hbm.at[pl.ds(rbase, _DEG_ROWS)])

    @pl.when((t == NS - 1) & (c == 1))
    def _wr_tail1():
        pltpu.sync_copy(out_v.at[pl.ds(0, 2960)], d1_hbm.at[pl.ds(rbase, 2960)])


_NW_B = 3136
_NW_PER = 25088


@functools.partial(
    pl.kernel,
    out_type=jax.ShapeDtypeStruct((N_EDGES,), jnp.float32),
    scratch_types=[
        pltpu.VMEM((N_NODES,), jnp.float32),
        pltpu.VMEM((N_NODES,), jnp.float32),
        pltpu.VMEM((_NW_B,), jnp.int32),
        pltpu.VMEM((_NW_B,), jnp.int32),
        pltpu.VMEM((_NW_B,), jnp.float32),
        pltpu.VMEM((_NW_B,), jnp.float32),
    ],
    **_MESH,
)
def _normw_kernel(src_hbm, dst_hbm, w_hbm, d0_hbm, d1_hbm, nw_hbm, d0_v,
                  d1_v, src_v, dst_v, w_v, o_v):
    wid = lax.axis_index("s") * NC + lax.axis_index("c")
    pltpu.sync_copy(d0_hbm, d0_v)
    pltpu.sync_copy(d1_hbm, d1_v)
    base = wid * _NW_PER
    n_blocks = jnp.where(wid < NC * NS - 1, _NW_PER // _NW_B, 7)

    def _inner(g, _):
        s = pl.ds(g * L, L)
        a = plsc.load_gather(d0_v, [src_v[s]])
        b = plsc.load_gather(d1_v, [dst_v[s]])
        o_v[s] = a * w_v[s] * b
        return _

    def _blk(j, _):
        off = base + j * _NW_B
        pltpu.sync_copy(src_hbm.at[pl.ds(off, _NW_B)], src_v)
        pltpu.sync_copy(dst_hbm.at[pl.ds(off, _NW_B)], dst_v)
        pltpu.sync_copy(w_hbm.at[pl.ds(off, _NW_B)], w_v)
        lax.fori_loop(0, _NW_B // L, _inner, None)
        pltpu.sync_copy(o_v, nw_hbm.at[pl.ds(off, _NW_B)])
        return _
    lax.fori_loop(0, n_blocks, _blk, None)

    @pl.when(wid == NC * NS - 1)
    def _tail():
        off = base + 7 * _NW_B
        pltpu.sync_copy(src_hbm.at[pl.ds(off, 320)], src_v.at[pl.ds(0, 320)])
        pltpu.sync_copy(dst_hbm.at[pl.ds(off, 320)], dst_v.at[pl.ds(0, 320)])
        pltpu.sync_copy(w_hbm.at[pl.ds(off, 320)], w_v.at[pl.ds(0, 320)])
        lax.fori_loop(0, 320 // L, _inner, None)
        pltpu.sync_copy(o_v.at[pl.ds(0, 320)], nw_hbm.at[pl.ds(off, 320)])


_LY_B = 80
_LY_EB = 2000
_LY_PER_TILE = N_EDGES // NS
_NQ_FULL = HALF // _LY_B


@functools.partial(
    pl.kernel,
    out_type=jax.ShapeDtypeStruct((N_NODES, D), jnp.float32),
    scratch_types=[
        pltpu.VMEM_SHARED((HALF, D), jnp.float32),
        pltpu.VMEM((_LY_EB,), jnp.int32),
        pltpu.VMEM((_LY_EB,), jnp.int32),
        pltpu.VMEM((_LY_EB,), jnp.float32),
        pltpu.VMEM((_LY_B, D), jnp.float32),
        pltpu.VMEM((_LY_B, D), jnp.float32),
        pltpu.VMEM((_LY_B, D), jnp.float32),
        pltpu.VMEM((_LY_B, D), jnp.float32),
        pltpu.VMEM((_LY_B,), jnp.int32),
        pltpu.VMEM((_LY_B,), jnp.int32),
        pltpu.SemaphoreType.DMA,
        pltpu.SemaphoreType.DMA,
        pltpu.SemaphoreType.DMA,
        pltpu.SemaphoreType.DMA,
    ],
    **_MESH,
)
def _layer_kernel(emb_hbm, srce_hbm, dste_hbm, nw_hbm, out_hbm, acc, src_v,
                  dst_v, nw_v, gb0, gb1, sb0, sb1, sx0, sx1, gsem0, gsem1,
                  ssem0, ssem1):
    c = lax.axis_index("c")
    t = lax.axis_index("s")
    lo = c * HALF
    zeros16 = jnp.zeros((L,), jnp.float32)

    for k in range(D // L):
        def _z(i, _, k=k):
            sb0[i, pl.ds(k * L, L)] = zeros16
            return _
        lax.fori_loop(0, _LY_B, _z, None)
    n_q = jnp.where(t < 9, 20, 19)

    def _zacc(i, _):
        q = t + i * NS

        @pl.when(q < _NQ_FULL)
        def _full():
            pltpu.sync_copy(sb0, acc.at[pl.ds(q * _LY_B, _LY_B)])

        @pl.when(q == _NQ_FULL)
        def _tail():
            pltpu.sync_copy(sb0.at[pl.ds(0, 40)],
                            acc.at[pl.ds(q * _LY_B, 40)])
        return _
    lax.fori_loop(0, n_q, _zacc, None)
    plsc.subcore_barrier()

    def _issue_g(off, gb, gsem):
        pltpu.async_copy(emb_hbm.at[dst_v.at[pl.ds(off, _LY_B)]], gb, gsem)

    def _wait_g(off, gb, gsem):
        pltpu.make_async_copy(emb_hbm.at[dst_v.at[pl.ds(off, _LY_B)]], gb,
                              gsem).wait()

    def _issue_s(sb, sx, ssem):
        pltpu.async_copy(sb, acc.at[sx], ssem, add=True)

    def _wait_s(sb, sx, ssem):
        pltpu.make_async_copy(sb, acc.at[sx], ssem).wait()

    def _scale_chunk(off, gb, sb, sx):
        for g in range(_LY_B // L):
            s16 = pl.ds(off + g * L, L)
            sv = src_v[s16]
            loc = sv - lo
            m = (loc >= 0) & (loc < HALF)
            sx[pl.ds(g * L, L)] = jnp.where(m, loc, sv & 16383)
            wv = jnp.where(m, nw_v[s16], 0.0)
            for u in range(L):
                w = wv[u]
                e = g * L + u
                for k in range(D // L):
                    ss = pl.ds(k * L, L)
                    sb[e, ss] = gb[e, ss] * w

    def _block(bj, _):
        eb = t * _LY_PER_TILE + bj * _LY_EB
        pltpu.sync_copy(srce_hbm.at[pl.ds(eb, _LY_EB)], src_v)
        pltpu.sync_copy(dste_hbm.at[pl.ds(eb, _LY_EB)], dst_v)
        pltpu.sync_copy(nw_hbm.at[pl.ds(eb, _LY_EB)], nw_v)
        _issue_g(0, gb0, gsem0)

        def _pair(i, _):
            o0 = 2 * i * _LY_B
            o1 = o0 + _LY_B
            o2 = o1 + _LY_B
            _wait_g(o0, gb0, gsem0)
            _issue_g(o1, gb1, gsem1)

            @pl.when(i >= 1)
            def _w0():
                _wait_s(sb0, sx0, ssem0)
            _scale_chunk(o0, gb0, sb0, sx0)
            _issue_s(sb0, sx0, ssem0)

            _wait_g(o1, gb1, gsem1)
            _issue_g(o2, gb0, gsem0)

            @pl.when(i >= 1)
            def _w1():
                _wait_s(sb1, sx1, ssem1)
            _scale_chunk(o1, gb1, sb1, sx1)
            _issue_s(sb1, sx1, ssem1)
            return _
        lax.fori_loop(0, 12, _pair, None)

        o24 = 24 * _LY_B
        _wait_g(o24, gb0, gsem0)
        _wait_s(sb0, sx0, ssem0)
        _scale_chunk(o24, gb0, sb0, sx0)
        _issue_s(sb0, sx0, ssem0)
        _wait_s(sb1, sx1, ssem1)
        _wait_s(sb0, sx0, ssem0)
        return _
    lax.fori_loop(0, _LY_PER_TILE // _LY_EB, _block, None)
    plsc.subcore_barrier()

    def _wb(i, _):
        q = t + i * NS

        @pl.when(q < _NQ_FULL)
        def _full():
            pltpu.sync_copy(acc.at[pl.ds(q * _LY_B, _LY_B)], gb0)
            pltpu.sync_copy(gb0, out_hbm.at[pl.ds(c * HALF + q * _LY_B,
                                                  _LY_B)])

        @pl.when(q == _NQ_FULL)
        def _tail():
            pltpu.sync_copy(acc.at[pl.ds(q * _LY_B, 40)],
                            gb0.at[pl.ds(0, 40)])
            pltpu.sync_copy(gb0.at[pl.ds(0, 40)],
                            out_hbm.at[pl.ds(c * HALF + q * _LY_B, 40)])
        return _
    lax.fori_loop(0, n_q, _wb, None)


def _mean_body(a, b, c, d, o):
    o[...] = (a[...] + b[...] + c[...] + d[...]) * 0.25


def _mean4(e0, e1, e2, e3):
    blk = pl.BlockSpec((2000, D), lambda i: (i, 0))
    return pl.pallas_call(
        _mean_body,
        grid=(N_NODES // 2000,),
        in_specs=[blk] * 4,
        out_specs=blk,
        out_shape=jax.ShapeDtypeStruct((N_NODES, D), jnp.float32),
    )(e0, e1, e2, e3)


def kernel(embedding_weight, edge_index, edge_weight):
    src = edge_index[0]
    dst = edge_index[1]
    nchunks = N_EDGES // 80
    d0, d1 = _deg_kernel(src.reshape(nchunks, 80), dst.reshape(nchunks, 80),
                         edge_weight.reshape(nchunks, 80))
    nw = _normw_kernel(src, dst, edge_weight, d0, d1)
    emb = embedding_weight
    embs = [emb]
    for _ in range(N_LAYERS):
        emb = _layer_kernel(emb, src, dst, nw)
        embs.append(emb)
    return _mean4(*embs)


# speedup vs baseline: 13.4575x; 1.1035x over previous
import functools

import jax
import jax.numpy as jnp
from jax import lax
from jax.experimental import pallas as pl
from jax.experimental.pallas import tpu as pltpu
from jax.experimental.pallas import tpu_sc as plsc

N_USERS = 30000
N_ITEMS = 19000
N_GENRES = 1000
N_NODES = 50000
N_EDGES = 800000
D = 64
N_LAYERS = 3

NC = 2
NS = 16
L = 16
HALF = N_NODES // NC
TRASH = 600
R_ACC = HALF + TRASH

_MESH = dict(
    mesh=plsc.VectorSubcoreMesh(core_axis_name="c", subcore_axis_name="s"),
    compiler_params=pltpu.CompilerParams(
        needs_layout_passes=False, use_tc_tiling_on_sc=False),
)


def _rsqrt_newton(x):
    bits = lax.bitcast_convert_type(x, jnp.int32)
    y = lax.bitcast_convert_type(
        jnp.int32(0x5F3759DF) - lax.shift_right_logical(bits, 1), jnp.float32)
    for _ in range(3):
        y = y * (1.5 - 0.5 * x * y * y)
    return jnp.where(x > 0.0, y, 0.0)


_DEG_B = 80
_DEG_RPB = 25
_DEG_ROWS = 3136


@functools.partial(
    pl.kernel,
    out_type=(jax.ShapeDtypeStruct((N_NODES,), jnp.float32),
              jax.ShapeDtypeStruct((N_NODES,), jnp.float32)),
    scratch_types=[
        pltpu.VMEM_SHARED((N_NODES,), jnp.float32),
        pltpu.VMEM((_DEG_RPB, _DEG_B), jnp.int32),
        pltpu.VMEM((_DEG_RPB, _DEG_B), jnp.float32),
        pltpu.VMEM((_DEG_ROWS,), jnp.float32),
        pltpu.VMEM((_DEG_ROWS,), jnp.float32),
        pltpu.SemaphoreType.DMA,
    ],
    **_MESH,
)
def _deg_kernel(srcr_hbm, dstr_hbm, wr_hbm, d0_hbm, d1_hbm, acc, idx_v, w_v,
                rd_v, out_v, ssem):
    c = lax.axis_index("c")
    t = lax.axis_index("s")
    zeros16 = jnp.zeros((L,), jnp.float32)

    def _z(i, _):
        rd_v[pl.ds(i * L, L)] = zeros16
        return _
    lax.fori_loop(0, _DEG_ROWS // L, _z, None)
    rbase = t * _DEG_ROWS

    @pl.when(t < NS - 1)
    def _z_full():
        pltpu.sync_copy(rd_v, acc.at[pl.ds(rbase, _DEG_ROWS)])

    @pl.when(t == NS - 1)
    def _z_tail():
        pltpu.sync_copy(rd_v.at[pl.ds(0, 2960)], acc.at[pl.ds(rbase, 2960)])
    plsc.subcore_barrier()

    rows_per_tile = N_EDGES // NS // _DEG_B

    def _block(b, _):
        row0 = t * rows_per_tile + b * _DEG_RPB

        @pl.when(c == 0)
        def _ld0():
            pltpu.sync_copy(srcr_hbm.at[pl.ds(row0, _DEG_RPB)], idx_v)

        @pl.when(c == 1)
        def _ld1():
            pltpu.sync_copy(dstr_hbm.at[pl.ds(row0, _DEG_RPB)], idx_v)
        pltpu.sync_copy(wr_hbm.at[pl.ds(row0, _DEG_RPB)], w_v)
        for j in range(_DEG_RPB):
            pltpu.async_copy(w_v.at[j], acc.at[idx_v.at[j]], ssem, add=True)
        for j in range(_DEG_RPB):
            pltpu.make_async_copy(w_v.at[j], acc.at[idx_v.at[j]], ssem).wait()
        return _
    lax.fori_loop(0, rows_per_tile // _DEG_RPB, _block, None)
    plsc.subcore_barrier()

    n_grp = jnp.where(t < NS - 1, _DEG_ROWS // L, 2960 // L)

    @pl.when(t < NS - 1)
    def _rd_full():
        pltpu.sync_copy(acc.at[pl.ds(rbase, _DEG_ROWS)], rd_v)

    @pl.when(t == NS - 1)
    def _rd_tail():
        pltpu.sync_copy(acc.at[pl.ds(rbase, 2960)], rd_v.at[pl.ds(0, 2960)])

    def _g(g, _):
        s = pl.ds(g * L, L)
        out_v[s] = _rsqrt_newton(rd_v[s])
        return _
    lax.fori_loop(0, n_grp, _g, None)

    @pl.when((t < NS - 1) & (c == 0))
    def _wr_full0():
        pltpu.sync_copy(out_v, d0_hbm.at[pl.ds(rbase, _DEG_ROWS)])

    @pl.when((t == NS - 1) & (c == 0))
    def _wr_tail0():
        pltpu.sync_copy(out_v.at[pl.ds(0, 2960)], d0_hbm.at[pl.ds(rbase, 2960)])

    @pl.when((t < NS - 1) & (c == 1))
    def _wr_full1():
        pltpu.sync_copy(out_v, d1_hbm.at[pl.ds(rbase, _DEG_ROWS)])

    @pl.when((t == NS - 1) & (c == 1))
    def _wr_tail1():
        pltpu.sync_copy(out_v.at[pl.ds(0, 2960)], d1_hbm.at[pl.ds(rbase, 2960)])


_NW_B = 3136
_NW_PER = 25088


@functools.partial(
    pl.kernel,
    out_type=jax.ShapeDtypeStruct((N_EDGES,), jnp.float32),
    scratch_types=[
        pltpu.VMEM((N_NODES,), jnp.float32),
        pltpu.VMEM((N_NODES,), jnp.float32),
        pltpu.VMEM((_NW_B,), jnp.int32),
        pltpu.VMEM((_NW_B,), jnp.int32),
        pltpu.VMEM((_NW_B,), jnp.float32),
        pltpu.VMEM((_NW_B,), jnp.float32),
    ],
    **_MESH,
)
def _normw_kernel(src_hbm, dst_hbm, w_hbm, d0_hbm, d1_hbm, nw_hbm, d0_v,
                  d1_v, src_v, dst_v, w_v, o_v):
    wid = lax.axis_index("s") * NC + lax.axis_index("c")
    pltpu.sync_copy(d0_hbm, d0_v)
    pltpu.sync_copy(d1_hbm, d1_v)
    base = wid * _NW_PER
    n_blocks = jnp.where(wid < NC * NS - 1, _NW_PER // _NW_B, 7)

    def _inner(g, _):
        s = pl.ds(g * L, L)
        a = plsc.load_gather(d0_v, [src_v[s]])
        b = plsc.load_gather(d1_v, [dst_v[s]])
        o_v[s] = a * w_v[s] * b
        return _

    def _blk(j, _):
        off = base + j * _NW_B
        pltpu.sync_copy(src_hbm.at[pl.ds(off, _NW_B)], src_v)
        pltpu.sync_copy(dst_hbm.at[pl.ds(off, _NW_B)], dst_v)
        pltpu.sync_copy(w_hbm.at[pl.ds(off, _NW_B)], w_v)
        lax.fori_loop(0, _NW_B // L, _inner, None)
        pltpu.sync_copy(o_v, nw_hbm.at[pl.ds(off, _NW_B)])
        return _
    lax.fori_loop(0, n_blocks, _blk, None)

    @pl.when(wid == NC * NS - 1)
    def _tail():
        off = base + 7 * _NW_B
        pltpu.sync_copy(src_hbm.at[pl.ds(off, 320)], src_v.at[pl.ds(0, 320)])
        pltpu.sync_copy(dst_hbm.at[pl.ds(off, 320)], dst_v.at[pl.ds(0, 320)])
        pltpu.sync_copy(w_hbm.at[pl.ds(off, 320)], w_v.at[pl.ds(0, 320)])
        lax.fori_loop(0, 320 // L, _inner, None)
        pltpu.sync_copy(o_v.at[pl.ds(0, 320)], nw_hbm.at[pl.ds(off, 320)])


_LY_B = 80
_LY_EB = 2000
_LY_PER_TILE = N_EDGES // NS
_D2 = D // 2


@functools.partial(
    pl.kernel,
    out_type=jax.ShapeDtypeStruct((2 * N_NODES, _D2), jnp.float32),
    scratch_types=[
        pltpu.VMEM_SHARED((N_NODES, _D2), jnp.float32),
        pltpu.VMEM((_LY_EB,), jnp.int32),
        pltpu.VMEM((_LY_EB,), jnp.int32),
        pltpu.VMEM((_LY_EB,), jnp.float32),
        pltpu.VMEM((_LY_B, _D2), jnp.float32),
        pltpu.VMEM((_LY_B, _D2), jnp.float32),
        pltpu.VMEM((_LY_B, _D2), jnp.float32),
        pltpu.VMEM((_LY_B, _D2), jnp.float32),
        pltpu.VMEM((_LY_B,), jnp.int32),
        pltpu.VMEM((_LY_B,), jnp.int32),
        pltpu.VMEM((_LY_B,), jnp.int32),
        pltpu.VMEM((_LY_B,), jnp.int32),
        pltpu.VMEM((_LY_B,), jnp.int32),
        pltpu.SemaphoreType.DMA,
        pltpu.SemaphoreType.DMA,
        pltpu.SemaphoreType.DMA,
        pltpu.SemaphoreType.DMA,
    ],
    **_MESH,
)
def _layer_kernel(emb_hbm, srce_hbm, dste_hbm, nw_hbm, out_hbm, acc, src_v,
                  dst_v, nw_v, gb0, gb1, sb0, sb1, gx0, gx1, sx0, sx1, wbx,
                  gsem0, gsem1, ssem0, ssem1):
    c = lax.axis_index("c")
    t = lax.axis_index("s")
    iota = lax.iota(jnp.int32, L)
    zeros16 = jnp.zeros((L,), jnp.float32)

    for k in range(_D2 // L):
        def _z(i, _, k=k):
            sb0[i, pl.ds(k * L, L)] = zeros16
            return _
        lax.fori_loop(0, _LY_B, _z, None)
    n_q = jnp.where(t < 1, 40, 39)

    def _zacc(i, _):
        q = t + i * NS
        pltpu.sync_copy(sb0, acc.at[pl.ds(q * _LY_B, _LY_B)])
        return _
    lax.fori_loop(0, n_q, _zacc, None)
    plsc.subcore_barrier()

    def _prep_g(off, gx):
        for g in range(_LY_B // L):
            s16 = pl.ds(off + g * L, L)
            gx[pl.ds(g * L, L)] = dst_v[s16] * 2 + c

    def _issue_g(gb, gx, gsem):
        pltpu.async_copy(emb_hbm.at[gx], gb, gsem)

    def _wait_g(gb, gx, gsem):
        pltpu.make_async_copy(emb_hbm.at[gx], gb, gsem).wait()

    def _issue_s(sb, sx, ssem):
        pltpu.async_copy(sb, acc.at[sx], ssem, add=True)

    def _wait_s(sb, sx, ssem):
        pltpu.make_async_copy(sb, acc.at[sx], ssem).wait()

    def _scale_chunk(off, gb, sb, sx):
        for g in range(_LY_B // L):
            s16 = pl.ds(off + g * L, L)
            sx[pl.ds(g * L, L)] = src_v[s16]
            wv = nw_v[s16]
            for u in range(L):
                w = wv[u]
                e = g * L + u
                for k in range(_D2 // L):
                    ss = pl.ds(k * L, L)
                    sb[e, ss] = gb[e, ss] * w

    def _block(bj, _):
        eb = t * _LY_PER_TILE + bj * _LY_EB
        pltpu.sync_copy(srce_hbm.at[pl.ds(eb, _LY_EB)], src_v)
        pltpu.sync_copy(dste_hbm.at[pl.ds(eb, _LY_EB)], dst_v)
        pltpu.sync_copy(nw_hbm.at[pl.ds(eb, _LY_EB)], nw_v)
        _prep_g(0, gx0)
        _issue_g(gb0, gx0, gsem0)

        def _pair(i, _):
            o0 = 2 * i * _LY_B
            o1 = o0 + _LY_B
            o2 = o1 + _LY_B
            _wait_g(gb0, gx0, gsem0)
            _prep_g(o1, gx1)
            _issue_g(gb1, gx1, gsem1)

            @pl.when(i >= 1)
            def _w0():
                _wait_s(sb0, sx0, ssem0)
            _scale_chunk(o0, gb0, sb0, sx0)
            _issue_s(sb0, sx0, ssem0)

            _wait_g(gb1, gx1, gsem1)
            _prep_g(o2, gx0)
            _issue_g(gb0, gx0, gsem0)

            @pl.when(i >= 1)
            def _w1():
                _wait_s(sb1, sx1, ssem1)
            _scale_chunk(o1, gb1, sb1, sx1)
            _issue_s(sb1, sx1, ssem1)
            return _
        lax.fori_loop(0, 12, _pair, None)

        o24 = 24 * _LY_B
        _wait_g(gb0, gx0, gsem0)
        _wait_s(sb0, sx0, ssem0)
        _scale_chunk(o24, gb0, sb0, sx0)
        _issue_s(sb0, sx0, ssem0)
        _wait_s(sb1, sx1, ssem1)
        _wait_s(sb0, sx0, ssem0)
        return _
    lax.fori_loop(0, _LY_PER_TILE // _LY_EB, _block, None)
    plsc.subcore_barrier()

    def _wb(i, _):
        q = t + i * NS
        r0 = q * _LY_B
        for g in range(_LY_B // L):
            wbx[pl.ds(g * L, L)] = (r0 + g * L + iota) * 2 + c
        pltpu.sync_copy(acc.at[pl.ds(r0, _LY_B)], gb0)
        pltpu.sync_copy(gb0, out_hbm.at[wbx])
        return _
    lax.fori_loop(0, n_q, _wb, None)


def _mean_body(a, b, c, d, o):
    o[...] = (a[...] + b[...] + c[...] + d[...]) * 0.25


def _mean4(e0, e1, e2, e3):
    blk = pl.BlockSpec((2000, D), lambda i: (i, 0))
    return pl.pallas_call(
        _mean_body,
        grid=(N_NODES // 2000,),
        in_specs=[blk] * 4,
        out_specs=blk,
        out_shape=jax.ShapeDtypeStruct((N_NODES, D), jnp.float32),
    )(e0, e1, e2, e3)


def kernel(embedding_weight, edge_index, edge_weight):
    src = edge_index[0]
    dst = edge_index[1]
    nchunks = N_EDGES // 80
    d0, d1 = _deg_kernel(src.reshape(nchunks, 80), dst.reshape(nchunks, 80),
                         edge_weight.reshape(nchunks, 80))
    nw = _normw_kernel(src, dst, edge_weight, d0, d1)
    embs = [embedding_weight]
    emb2 = embedding_weight.reshape(2 * N_NODES, D // 2)
    for _ in range(N_LAYERS):
        emb2 = _layer_kernel(emb2, src, dst, nw)
        embs.append(emb2.reshape(N_NODES, D))
    return _mean4(*embs)
